# baseline (speedup 1.0000x reference)
import jax
import jax.numpy as jnp
from jax.experimental import pallas as pl

ITERS = 5
C = 64


def kernel(x, edge_index, edge_attr, W0, b0, Wm1, bm1, Wm2, bm2, Wu1, bu1, Wu2, bu2, Wr1, br1, Wr2, br2):
    n = x.shape[0]
    src = edge_index[0]
    dst = edge_index[1]

    cnt = jax.ops.segment_sum(jnp.ones((src.shape[0],), jnp.float32), dst, num_segments=n)
    cnt_c = jnp.clip(cnt, 1.0)[:, None]
    mask = (cnt > 0)[:, None]

    Wm1_d, Wm1_s, Wm1_e = Wm1[:C], Wm1[C:2 * C], Wm1[2 * C:]
    ea_m = edge_attr @ Wm1_e + bm1

    h = x @ W0 + b0
    for _ in range(ITERS):
        hr = jax.nn.relu(h)
        A = hr @ Wm1_d
        B = hr @ Wm1_s
        t = jax.nn.relu(A[dst] + B[src] + ea_m)
        m = t @ Wm2 + bm2
        s = jax.ops.segment_sum(m, dst, num_segments=n)
        mean = s / cnt_c
        mean_sq = jax.ops.segment_sum(m * m, dst, num_segments=n) / cnt_c
        var = jax.nn.relu(mean_sq - mean * mean)
        std = jnp.where(mask, jnp.sqrt(var + 1e-5), 0.0)
        mn = jnp.where(mask, jax.ops.segment_min(m, dst, num_segments=n), 0.0)
        mx = jnp.where(mask, jax.ops.segment_max(m, dst, num_segments=n), 0.0)
        agg = jnp.concatenate([std, mn, mx, mean], axis=1)
        z = jnp.concatenate([agg, hr], axis=1)
        h = jax.nn.relu(z @ Wu1 + bu1) @ Wu2 + bu2

    Wr1_s, Wr1_d, Wr1_e = Wr1[:C], Wr1[C:2 * C], Wr1[2 * C:]
    ea_r = edge_attr @ Wr1_e + br1
    P = h @ Wr1_s
    Q = h @ Wr1_d
    q = jax.nn.relu(P[src] + Q[dst] + ea_r) @ Wr2 + br2
    return q


# speedup vs baseline: 2.6595x; 2.6595x over previous
import functools
import jax
import jax.numpy as jnp
import numpy as np
from jax import lax
from jax.experimental import pallas as pl
from jax.experimental.pallas import tpu as pltpu
from jax.experimental.pallas import tpu_sc as plsc

N = 10000
E = 320000
C = 64
ITERS = 5
NTILES = 32
NPT = 320
NPAD = NTILES * NPT
EPT = E // NTILES
G = 80
NEG = -3.0e38
POS = 3.0e38


def _mm_call(fn, out_shapes, grid, in_specs, out_specs, *args):
    return pl.pallas_call(
        fn, out_shape=out_shapes, grid=grid, in_specs=in_specs,
        out_specs=out_specs)(*args)


def _full(shape):
    return pl.BlockSpec(shape, lambda i: (0,) * len(shape))


def _rows(bs, ncols):
    return pl.BlockSpec((bs, ncols), lambda i: (i, 0))


def _prep_edges_k(ea_ref, we_m_ref, bm1_ref, we_r_ref, br1_ref, om_ref, or_ref):
    a0 = ea_ref[:, 0:1]
    a1 = ea_ref[:, 1:2]
    om_ref[...] = a0 * we_m_ref[0:1, :] + a1 * we_m_ref[1:2, :] + bm1_ref[...]
    or_ref[...] = a0 * we_r_ref[0:1, :] + a1 * we_r_ref[1:2, :] + br1_ref[...]


def prep_edges(edge_attr, we_m, bm1, we_r, br1):
    eb = 8000
    return _mm_call(
        _prep_edges_k,
        [jax.ShapeDtypeStruct((E, C), jnp.float32)] * 2,
        (E // eb,),
        [_rows(eb, 2), _full((2, C)), _full((1, C)), _full((2, C)),
         _full((1, C))],
        [_rows(eb, C), _rows(eb, C)],
        edge_attr, we_m, bm1, we_r, br1)


def _init_k(x_ref, w_ref, b_ref, hr_ref):
    h = jnp.dot(x_ref[...], w_ref[...], preferred_element_type=jnp.float32)
    hr_ref[...] = jnp.maximum(h + b_ref[...], 0.0)


def init_nodes(x, W0, b0):
    nb = 2000
    return _mm_call(
        _init_k, jax.ShapeDtypeStruct((N, C), jnp.float32), (N // nb,),
        [_rows(nb, 3), _full((3, C)), _full((1, C))], _rows(nb, C),
        x, W0, b0)


def _proj_k(y_ref, wl_ref, wr_ref, a_ref, b_ref):
    y = y_ref[...]
    a_ref[...] = jnp.dot(y, wl_ref[...], preferred_element_type=jnp.float32)
    b_ref[...] = jnp.dot(y, wr_ref[...], preferred_element_type=jnp.float32)


def proj_nodes(y, wl, wr):
    nb = 2000
    return _mm_call(
        _proj_k, [jax.ShapeDtypeStruct((N, C), jnp.float32)] * 2, (N // nb,),
        [_rows(nb, C), _full((C, C)), _full((C, C))],
        [_rows(nb, C), _rows(nb, C)],
        y, wl, wr)


def _msg_k(t_ref, w_ref, m_ref):
    m_ref[...] = jnp.dot(t_ref[...], w_ref[...],
                         preferred_element_type=jnp.float32)


def msg_matmul(t, Wm2):
    eb = 8000
    return _mm_call(
        _msg_k, jax.ShapeDtypeStruct((E, C), jnp.float32), (E // eb,),
        [_rows(eb, C), _full((C, C))], _rows(eb, C), t, Wm2)


def _update_k(agg_ref, cnt_ref, hr_ref, wu1_ref, bu1_ref, wu2_ref, bu2_ref,
              bm2_ref, h_ref, hro_ref):
    cnt = cnt_ref[...]
    cntc = jnp.maximum(cnt, 1.0)
    mask = cnt > 0.0
    bm2 = bm2_ref[...]
    s = agg_ref[0]
    sq = agg_ref[1]
    mn = agg_ref[2]
    mx = agg_ref[3]
    mean_m = s / cntc
    var = jnp.maximum(sq / cntc - mean_m * mean_m, 0.0)
    std = jnp.where(mask, jnp.sqrt(var + 1e-5), 0.0)
    mean = jnp.where(mask, mean_m + bm2, 0.0)
    mnb = jnp.where(mask, mn + bm2, 0.0)
    mxb = jnp.where(mask, mx + bm2, 0.0)
    hr = hr_ref[...]
    z = (jnp.dot(std, wu1_ref[0:C, :], preferred_element_type=jnp.float32)
         + jnp.dot(mnb, wu1_ref[C:2 * C, :], preferred_element_type=jnp.float32)
         + jnp.dot(mxb, wu1_ref[2 * C:3 * C, :],
                   preferred_element_type=jnp.float32)
         + jnp.dot(mean, wu1_ref[3 * C:4 * C, :],
                   preferred_element_type=jnp.float32)
         + jnp.dot(hr, wu1_ref[4 * C:5 * C, :],
                   preferred_element_type=jnp.float32)
         + bu1_ref[...])
    h = jnp.dot(jnp.maximum(z, 0.0), wu2_ref[...],
                preferred_element_type=jnp.float32) + bu2_ref[...]
    h_ref[...] = h
    hro_ref[...] = jnp.maximum(h, 0.0)


def update_nodes(agg, cnt2d, hr, Wu1, bu1, Wu2, bu2, bm2):
    nb = 2000
    return _mm_call(
        _update_k, [jax.ShapeDtypeStruct((N, C), jnp.float32)] * 2,
        (N // nb,),
        [pl.BlockSpec((4, nb, C), lambda i: (0, i, 0)),
         _rows(nb, 1), _rows(nb, C), _full((5 * C, C)), _full((1, C)),
         _full((C, C)), _full((1, C)), _full((1, C))],
        [_rows(nb, C), _rows(nb, C)],
        agg, cnt2d, hr, Wu1, bu1, Wu2, bu2, bm2)


_MESH = plsc.VectorSubcoreMesh(core_axis_name="c", subcore_axis_name="s")
_SC_PARAMS = pltpu.CompilerParams(use_tc_tiling_on_sc=False)


def _wid():
    return lax.axis_index("s") * 2 + lax.axis_index("c")


@functools.partial(
    pl.kernel, mesh=_MESH, compiler_params=_SC_PARAMS,
    out_type=jax.ShapeDtypeStruct((E, C), jnp.float32),
    scratch_types=[
        pltpu.VMEM((G,), jnp.int32),
        pltpu.VMEM((G,), jnp.int32),
        pltpu.VMEM((G, C), jnp.float32),
        pltpu.VMEM((G, C), jnp.float32),
        pltpu.VMEM((G, C), jnp.float32),
        pltpu.VMEM((G, C), jnp.float32),
        pltpu.SemaphoreType.DMA,
        pltpu.SemaphoreType.DMA,
    ])
def sc_gather(a_hbm, b_hbm, src_hbm, dst_hbm, ea_hbm, t_hbm,
              dstv, srcv, rowsa, rowsb, eav, tv, sema, semb):
    ebase = _wid() * EPT

    def chunk(k, _):
        cb = ebase + k * G
        pltpu.sync_copy(dst_hbm.at[pl.ds(cb, G)], dstv)
        pltpu.sync_copy(src_hbm.at[pl.ds(cb, G)], srcv)
        cpa = pltpu.async_copy(a_hbm.at[dstv], rowsa, sema)
        cpb = pltpu.async_copy(b_hbm.at[srcv], rowsb, semb)
        pltpu.sync_copy(ea_hbm.at[pl.ds(cb, G), :], eav)
        cpa.wait()
        cpb.wait()

        def row(j, _):
            for c in range(4):
                sl = pl.ds(c * 16, 16)
                tv[j, sl] = jnp.maximum(
                    rowsa[j, sl] + rowsb[j, sl] + eav[j, sl], 0.0)
            return _

        lax.fori_loop(0, G, row, None)
        pltpu.sync_copy(tv, t_hbm.at[pl.ds(cb, G), :])
        return _

    lax.fori_loop(0, EPT // G, chunk, None)


@functools.partial(
    pl.kernel, mesh=_MESH, compiler_params=_SC_PARAMS,
    out_type=jax.ShapeDtypeStruct((4, NPAD, C), jnp.float32),
    scratch_types=[
        pltpu.VMEM((48,), jnp.int32),
        pltpu.VMEM((G,), jnp.int32),
        pltpu.VMEM((G,), jnp.int32),
        pltpu.VMEM((G, C), jnp.float32),
        pltpu.VMEM((4, NPT + 1, C), jnp.float32),
        pltpu.SemaphoreType.DMA,
    ])
def sc_scatter(m_hbm, order_hbm, dsts_hbm, bounds_hbm, agg_hbm,
               bv, idxv, dstv, rowsm, acc, sem):
    w = _wid()
    nbase = w * NPT
    pltpu.sync_copy(bounds_hbm.at[pl.ds(0, 48)], bv)
    lov = bv[pl.ds(w, 16)]
    lo = lov[0]
    hi = lov[1]
    lo_a = (lo // 8) * 8
    nchunks = (hi - lo_a + (G - 1)) // G

    def initrow(n, _):
        for a in range(4):
            for c in range(4):
                sl = pl.ds(c * 16, 16)
                if a == 0 or a == 1:
                    acc[a, n, sl] = jnp.zeros((16,), jnp.float32)
                elif a == 2:
                    acc[a, n, sl] = jnp.full((16,), POS, jnp.float32)
                else:
                    acc[a, n, sl] = jnp.full((16,), NEG, jnp.float32)
        return _

    lax.fori_loop(0, NPT + 1, initrow, None)

    def chunk(k, _):
        cb = lo_a + k * G
        pltpu.sync_copy(order_hbm.at[pl.ds(cb, G)], idxv)
        pltpu.sync_copy(dsts_hbm.at[pl.ds(cb, G)], dstv)
        pltpu.async_copy(m_hbm.at[idxv], rowsm, sem).wait()

        def grp(g, _):
            dvec = dstv[pl.ds(g * 16, 16)]
            for lane in range(16):
                j = g * 16 + lane
                e = cb + j
                valid = jnp.logical_and(e >= lo, e < hi)
                o = jnp.where(valid, dvec[lane] - nbase, NPT)
                for c in range(4):
                    sl = pl.ds(c * 16, 16)
                    mv = rowsm[j, sl]
                    acc[0, o, sl] = acc[0, o, sl] + mv
                    acc[1, o, sl] = acc[1, o, sl] + mv * mv
                    acc[2, o, sl] = jnp.minimum(acc[2, o, sl], mv)
                    acc[3, o, sl] = jnp.maximum(acc[3, o, sl], mv)
            return _

        lax.fori_loop(0, G // 16, grp, None)
        return _

    lax.fori_loop(0, nchunks, chunk, None)
    for a in range(4):
        pltpu.sync_copy(acc.at[a, pl.ds(0, NPT), :],
                        agg_hbm.at[a, pl.ds(nbase, NPT), :])


def _readout_k(t_ref, w_ref, b_ref, q_ref):
    q_ref[...] = jnp.dot(t_ref[...], w_ref[...],
                         preferred_element_type=jnp.float32) + b_ref[...]


def readout_matmul(t_r, Wr2, br2):
    eb = 8000
    return _mm_call(
        _readout_k, jax.ShapeDtypeStruct((E, 1), jnp.float32), (E // eb,),
        [_rows(eb, C), _full((C, 1)), _full((1, 1))], _rows(eb, 1),
        t_r, Wr2, br2)


def kernel(x, edge_index, edge_attr, W0, b0, Wm1, bm1, Wm2, bm2, Wu1, bu1,
           Wu2, bu2, Wr1, br1, Wr2, br2):
    src = edge_index[0].astype(jnp.int32)
    dst = edge_index[1].astype(jnp.int32)

    iota = lax.iota(jnp.int32, E)
    dst_sorted, order = lax.sort((dst, iota), num_keys=1)
    tile_edges = jnp.searchsorted(dst_sorted,
                                  jnp.arange(0, NPAD + 1, NPT,
                                             dtype=jnp.int32)).astype(jnp.int32)
    bounds = jnp.zeros((48,), jnp.int32).at[:NTILES + 1].set(tile_edges)
    node_starts = jnp.searchsorted(
        dst_sorted, jnp.arange(N + 1, dtype=jnp.int32)).astype(jnp.int32)
    cnt2d = jnp.diff(node_starts).astype(jnp.float32)[:, None]
    order_p = jnp.concatenate([order, jnp.zeros((G,), jnp.int32)])
    dsts_p = jnp.concatenate([dst_sorted, jnp.zeros((G,), jnp.int32)])

    b0r = b0[None, :]
    bm1r = bm1[None, :]
    bu1r = bu1[None, :]
    bu2r = bu2[None, :]
    bm2r = bm2[None, :]
    br1r = br1[None, :]

    ea_m, ea_r = prep_edges(edge_attr, Wm1[2 * C:], bm1r, Wr1[2 * C:], br1r)
    hr = init_nodes(x, W0, b0r)

    h = hr
    for _ in range(ITERS):
        A, B = proj_nodes(hr, Wm1[:C], Wm1[C:2 * C])
        t = sc_gather(A, B, src, dst, ea_m)
        m = msg_matmul(t, Wm2)
        agg = sc_scatter(m, order_p, dsts_p, bounds)
        h, hr = update_nodes(agg, cnt2d, hr, Wu1, bu1r, Wu2, bu2r, bm2r)

    P, Q = proj_nodes(h, Wr1[:C], Wr1[C:2 * C])
    t_r = sc_gather(Q, P, src, dst, ea_r)
    return readout_matmul(t_r, Wr2, br2[None, :])
